# Initial kernel scaffold; baseline (speedup 1.0000x reference)
import jax
import jax.numpy as jnp
from jax.experimental import pallas as pl


def kernel(x, edge_index, W1, a_src1, a_dst1, b1, W2, a_src2, a_dst2, b2):
    raise NotImplementedError("write your pallas kernel here")


# speedup vs baseline: 36.0344x; 36.0344x over previous
import functools

import jax
import jax.numpy as jnp
from jax import lax
from jax.experimental import pallas as pl
from jax.experimental.pallas import tpu as pltpu
from jax.experimental.pallas import tpu_sc as plsc

N = 10000
NPAD = 10240
CHUNK = 128
NWORKERS = 32
ROWBLK = 1024

_GATHER_DNUMS = lax.GatherDimensionNumbers(
    offset_dims=(), collapsed_slice_dims=(0,), start_index_map=(0,))


def _lane_bcast(v16, l):
    idx = jnp.full((16, 1), l, dtype=jnp.int32)
    return lax.gather(v16, idx, dimension_numbers=_GATHER_DNUMS,
                      slice_sizes=(1,),
                      mode=lax.GatherScatterMode.PROMISE_IN_BOUNDS)


def _proj_body(x_ref, w_ref, o_ref, mx_ref):
    o = jnp.dot(x_ref[...], w_ref[...], preferred_element_type=jnp.float32)
    o_ref[...] = o

    @pl.when(pl.program_id(0) == 0)
    def _():
        mx_ref[...] = jnp.full_like(mx_ref, -1e30)

    mx_ref[...] = jnp.maximum(mx_ref[...], jnp.max(o, axis=0, keepdims=True))


def _proj(xp, wext):
    k = xp.shape[1]
    m = wext.shape[1]
    grid = NPAD // ROWBLK
    return pl.pallas_call(
        _proj_body,
        grid=(grid,),
        in_specs=[pl.BlockSpec((ROWBLK, k), lambda i: (i, 0)),
                  pl.BlockSpec((k, m), lambda i: (0, 0))],
        out_specs=[pl.BlockSpec((ROWBLK, m), lambda i: (i, 0)),
                   pl.BlockSpec((1, m), lambda i: (0, 0))],
        out_shape=[jax.ShapeDtypeStruct((NPAD, m), jnp.float32),
                   jax.ShapeDtypeStruct((1, m), jnp.float32)],
    )(xp, wext)


def _combine_proj_body(acc_ref, den_ref, b_ref, w_ref, o_ref, mx_ref):
    a = acc_ref[0] + acc_ref[1]
    dn = den_ref[0] + den_ref[1]
    o1 = a / (dn + 1e-16) + b_ref[...]
    h2 = jnp.where(o1 > 0.0, o1, jnp.exp(o1) - 1.0)
    o = jnp.dot(h2, w_ref[...], preferred_element_type=jnp.float32)
    o_ref[...] = o

    @pl.when(pl.program_id(0) == 0)
    def _():
        mx_ref[...] = jnp.full_like(mx_ref, -1e30)

    mx_ref[...] = jnp.maximum(mx_ref[...], jnp.max(o, axis=0, keepdims=True))


def _combine_proj(acc, den3, b, wext):
    c = acc.shape[2]
    m = wext.shape[1]
    grid = NPAD // ROWBLK
    return pl.pallas_call(
        _combine_proj_body,
        grid=(grid,),
        in_specs=[pl.BlockSpec((2, ROWBLK, c), lambda i: (0, i, 0)),
                  pl.BlockSpec((2, ROWBLK, 1), lambda i: (0, i, 0)),
                  pl.BlockSpec((1, c), lambda i: (0, 0)),
                  pl.BlockSpec((c, m), lambda i: (0, 0))],
        out_specs=[pl.BlockSpec((ROWBLK, m), lambda i: (i, 0)),
                   pl.BlockSpec((1, m), lambda i: (0, 0))],
        out_shape=[jax.ShapeDtypeStruct((NPAD, m), jnp.float32),
                   jax.ShapeDtypeStruct((1, m), jnp.float32)],
    )(acc, den3, b, wext)


def _final_body(acc_ref, den_ref, b_ref, o_ref):
    a = acc_ref[0] + acc_ref[1]
    dn = den_ref[0] + den_ref[1]
    o_ref[...] = a / (dn + 1e-16) + b_ref[...]


def _final(acc, den3, b):
    c = acc.shape[2]
    grid = NPAD // ROWBLK
    return pl.pallas_call(
        _final_body,
        grid=(grid,),
        in_specs=[pl.BlockSpec((2, ROWBLK, c), lambda i: (0, i, 0)),
                  pl.BlockSpec((2, ROWBLK, 1), lambda i: (0, i, 0)),
                  pl.BlockSpec((1, c), lambda i: (0, 0))],
        out_specs=pl.BlockSpec((ROWBLK, c), lambda i: (i, 0)),
        out_shape=jax.ShapeDtypeStruct((NPAD, c), jnp.float32),
    )(acc, den3, b)


def _make_edge_kernel(C, EP):
    n_chunks = EP // CHUNK // NWORKERS
    rpt = NPAD // 16
    mesh = plsc.VectorSubcoreMesh(core_axis_name="c", subcore_axis_name="s")

    @functools.partial(
        pl.kernel,
        mesh=mesh,
        compiler_params=pltpu.CompilerParams(needs_layout_passes=False,
                                             use_tc_tiling_on_sc=False),
        out_type=[jax.ShapeDtypeStruct((2 * NPAD, C), jnp.float32),
                  jax.ShapeDtypeStruct((2 * NPAD,), jnp.float32)],
        scratch_types=[
            pltpu.VMEM((NPAD,), jnp.float32),
            pltpu.VMEM((NPAD,), jnp.float32),
            pltpu.VMEM((16,), jnp.float32),
            pltpu.VMEM((CHUNK,), jnp.int32),
            pltpu.VMEM((CHUNK,), jnp.int32),
            pltpu.VMEM((CHUNK,), jnp.float32),
            pltpu.VMEM((CHUNK, C), jnp.float32),
            pltpu.VMEM_SHARED((NPAD, C), jnp.float32),
            pltpu.VMEM_SHARED((NPAD,), jnp.float32),
            pltpu.SemaphoreType.DMA,
        ],
    )
    def edge_kernel(src_hbm, dst_hbm, gmax_hbm, als_hbm, ald_hbm, h_hbm,
                    acc_out, den_out,
                    als_v, ald_v, gmax_v, src_v, dst_v, w_v, rows_v,
                    acc_s, den_s, sem):
        cid = lax.axis_index("c")
        sid = lax.axis_index("s")
        wid = sid * 2 + cid

        pltpu.sync_copy(als_hbm, als_v)
        pltpu.sync_copy(ald_hbm, ald_v)
        pltpu.sync_copy(gmax_hbm, gmax_v)

        z16 = jnp.zeros((16,), jnp.float32)

        def zrow(r, carry):
            for c in range(C // 16):
                rows_v[r, pl.ds(c * 16, 16)] = z16
            return carry

        lax.fori_loop(0, CHUNK, zrow, 0)

        def zw(r, carry):
            w_v[pl.ds(r * 16, 16)] = z16
            return carry

        lax.fori_loop(0, CHUNK // 16, zw, 0)

        row0 = sid * rpt
        for b in range(rpt // CHUNK):
            pltpu.sync_copy(rows_v, acc_s.at[pl.ds(row0 + b * CHUNK, CHUNK)])
            pltpu.sync_copy(w_v, den_s.at[pl.ds(row0 + b * CHUNK, CHUNK)])
        plsc.subcore_barrier()

        gv = gmax_v[...]
        base0 = wid * n_chunks * CHUNK

        def body(i, carry):
            base = base0 + i * CHUNK
            pltpu.sync_copy(src_hbm.at[pl.ds(base, CHUNK)], src_v)
            pltpu.sync_copy(dst_hbm.at[pl.ds(base, CHUNK)], dst_v)
            pltpu.async_copy(h_hbm.at[src_v], rows_v, sem).wait()
            for j in range(CHUNK // 16):
                s16 = src_v[pl.ds(j * 16, 16)]
                d16 = dst_v[pl.ds(j * 16, 16)]
                e = plsc.load_gather(als_v, [s16]) + plsc.load_gather(ald_v, [d16])
                e = jnp.where(e > 0.0, e, e * 0.2) - gv
                w16 = jnp.exp(e)
                w_v[pl.ds(j * 16, 16)] = w16
                for l in range(16):
                    wb = _lane_bcast(w16, l)
                    r = j * 16 + l
                    for c in range(C // 16):
                        rows_v[r, pl.ds(c * 16, 16)] = (
                            rows_v[r, pl.ds(c * 16, 16)] * wb)
            pltpu.sync_copy(rows_v, acc_s.at[dst_v], add=True)
            pltpu.sync_copy(w_v, den_s.at[dst_v], add=True)
            return carry

        lax.fori_loop(0, n_chunks, body, 0)
        plsc.subcore_barrier()

        out0 = cid * NPAD + row0
        pltpu.sync_copy(acc_s.at[pl.ds(row0, rpt)], acc_out.at[pl.ds(out0, rpt)])
        pltpu.sync_copy(den_s.at[pl.ds(row0, rpt)], den_out.at[pl.ds(out0, rpt)])

    return edge_kernel


def _gat_layer(edge_args, hext, mx, C, cols):
    srcf, dstf, ep = edge_args
    htab = hext[:, :C]
    als = hext[:, cols[0]].at[N:].set(-1e30)
    ald = hext[:, cols[1]]
    m = mx[0, cols[0]] + mx[0, cols[1]]
    g = jnp.where(m > 0.0, m, 0.2 * m)
    gvec = jnp.full((16,), g, jnp.float32)
    ek = _make_edge_kernel(C, ep)
    accf, denf = ek(srcf, dstf, gvec, als, ald, htab)
    return accf.reshape(2, NPAD, C), denf.reshape(2, NPAD, 1)


def kernel(x, edge_index, W1, a_src1, a_dst1, b1, W2, a_src2, a_dst2, b2):
    f32 = jnp.float32
    src = edge_index[0].astype(jnp.int32)
    dst = edge_index[1].astype(jnp.int32)
    loop = jnp.arange(N, dtype=jnp.int32)
    etot = src.shape[0] + N
    ep = -(-etot // (NWORKERS * CHUNK)) * (NWORKERS * CHUNK)
    padlen = ep - etot
    padv = jnp.full((padlen,), N, jnp.int32)
    srcf = jnp.concatenate([src, loop, padv])
    dstf = jnp.concatenate([dst, loop, padv])
    edge_args = (srcf, dstf, ep)

    d_in = x.shape[1]
    hid = W1.shape[1]
    out_ch = W2.shape[1]

    xp = jnp.zeros((NPAD, d_in), f32).at[:N].set(x.astype(f32))
    w1e = jnp.concatenate(
        [W1, (W1 @ a_src1[0])[:, None], (W1 @ a_dst1[0])[:, None],
         jnp.zeros((d_in, d_in - hid - 2), f32)], axis=1)
    hext1, mx1 = _proj(xp, w1e)
    acc1, den1 = _gat_layer(edge_args, hext1, mx1, hid, (hid, hid + 1))

    w2cols = max(out_ch + 2, 8)
    w2e = jnp.concatenate(
        [W2, (W2 @ a_src2[0])[:, None], (W2 @ a_dst2[0])[:, None],
         jnp.zeros((hid, w2cols - out_ch - 2), f32)], axis=1)
    hext2, mx2 = _combine_proj(acc1, den1, b1.reshape(1, hid), w2e)
    acc2, den2 = _gat_layer(edge_args, hext2, mx2, out_ch, (out_ch, out_ch + 1))

    out = _final(acc2, den2, b2.reshape(1, out_ch))
    return out[:N]


# speedup vs baseline: 58.8961x; 1.6344x over previous
import functools

import jax
import jax.numpy as jnp
from jax import lax
from jax.experimental import pallas as pl
from jax.experimental.pallas import tpu as pltpu
from jax.experimental.pallas import tpu_sc as plsc

N = 10000
NPAD = 10240
CHUNK = 128
NWORKERS = 32
ROWBLK = 1024

_GATHER_DNUMS = lax.GatherDimensionNumbers(
    offset_dims=(), collapsed_slice_dims=(0,), start_index_map=(0,))


def _lane_bcast(v16, l):
    idx = jnp.full((16, 1), l, dtype=jnp.int32)
    return lax.gather(v16, idx, dimension_numbers=_GATHER_DNUMS,
                      slice_sizes=(1,),
                      mode=lax.GatherScatterMode.PROMISE_IN_BOUNDS)


def _proj_body(x_ref, w_ref, o_ref, mx_ref):
    o = jnp.dot(x_ref[...], w_ref[...], preferred_element_type=jnp.float32)
    o_ref[...] = o

    @pl.when(pl.program_id(0) == 0)
    def _():
        mx_ref[...] = jnp.full_like(mx_ref, -1e30)

    mx_ref[...] = jnp.maximum(mx_ref[...], jnp.max(o, axis=0, keepdims=True))


def _proj(xp, wext):
    k = xp.shape[1]
    m = wext.shape[1]
    grid = NPAD // ROWBLK
    return pl.pallas_call(
        _proj_body,
        grid=(grid,),
        in_specs=[pl.BlockSpec((ROWBLK, k), lambda i: (i, 0)),
                  pl.BlockSpec((k, m), lambda i: (0, 0))],
        out_specs=[pl.BlockSpec((ROWBLK, m), lambda i: (i, 0)),
                   pl.BlockSpec((1, m), lambda i: (0, 0))],
        out_shape=[jax.ShapeDtypeStruct((NPAD, m), jnp.float32),
                   jax.ShapeDtypeStruct((1, m), jnp.float32)],
    )(xp, wext)


def _combine_proj_body(acc_ref, den_ref, b_ref, w_ref, o_ref, mx_ref):
    a = acc_ref[0] + acc_ref[1]
    dn = den_ref[0] + den_ref[1]
    o1 = a / (dn + 1e-16) + b_ref[...]
    h2 = jnp.where(o1 > 0.0, o1, jnp.exp(o1) - 1.0)
    o = jnp.dot(h2, w_ref[...], preferred_element_type=jnp.float32)
    o_ref[...] = o

    @pl.when(pl.program_id(0) == 0)
    def _():
        mx_ref[...] = jnp.full_like(mx_ref, -1e30)

    mx_ref[...] = jnp.maximum(mx_ref[...], jnp.max(o, axis=0, keepdims=True))


def _combine_proj(acc, den3, b, wext):
    c = acc.shape[2]
    m = wext.shape[1]
    grid = NPAD // ROWBLK
    return pl.pallas_call(
        _combine_proj_body,
        grid=(grid,),
        in_specs=[pl.BlockSpec((2, ROWBLK, c), lambda i: (0, i, 0)),
                  pl.BlockSpec((2, ROWBLK, 1), lambda i: (0, i, 0)),
                  pl.BlockSpec((1, c), lambda i: (0, 0)),
                  pl.BlockSpec((c, m), lambda i: (0, 0))],
        out_specs=[pl.BlockSpec((ROWBLK, m), lambda i: (i, 0)),
                   pl.BlockSpec((1, m), lambda i: (0, 0))],
        out_shape=[jax.ShapeDtypeStruct((NPAD, m), jnp.float32),
                   jax.ShapeDtypeStruct((1, m), jnp.float32)],
    )(acc, den3, b, wext)


def _final_body(acc_ref, den_ref, b_ref, o_ref):
    a = acc_ref[0] + acc_ref[1]
    dn = den_ref[0] + den_ref[1]
    o_ref[...] = a / (dn + 1e-16) + b_ref[...]


def _final(acc, den3, b):
    c = acc.shape[2]
    grid = NPAD // ROWBLK
    return pl.pallas_call(
        _final_body,
        grid=(grid,),
        in_specs=[pl.BlockSpec((2, ROWBLK, c), lambda i: (0, i, 0)),
                  pl.BlockSpec((2, ROWBLK, 1), lambda i: (0, i, 0)),
                  pl.BlockSpec((1, c), lambda i: (0, 0))],
        out_specs=pl.BlockSpec((ROWBLK, c), lambda i: (i, 0)),
        out_shape=jax.ShapeDtypeStruct((NPAD, c), jnp.float32),
    )(acc, den3, b)


def _make_edge_kernel(C, EP):
    n_chunks = EP // CHUNK // NWORKERS
    n3 = n_chunks // 3
    rpt = NPAD // 16
    mesh = plsc.VectorSubcoreMesh(core_axis_name="c", subcore_axis_name="s")

    @functools.partial(
        pl.kernel,
        mesh=mesh,
        compiler_params=pltpu.CompilerParams(needs_layout_passes=False,
                                             use_tc_tiling_on_sc=False),
        out_type=[jax.ShapeDtypeStruct((2 * NPAD, C), jnp.float32),
                  jax.ShapeDtypeStruct((2 * NPAD,), jnp.float32)],
        scratch_types=[
            pltpu.VMEM((NPAD,), jnp.float32),
            pltpu.VMEM((NPAD,), jnp.float32),
            pltpu.VMEM((16,), jnp.float32),
            pltpu.VMEM((n_chunks, CHUNK), jnp.int32),
            pltpu.VMEM((n_chunks, CHUNK), jnp.int32),
            pltpu.VMEM((CHUNK,), jnp.float32),
            pltpu.VMEM((CHUNK,), jnp.float32),
            pltpu.VMEM((CHUNK,), jnp.float32),
            pltpu.VMEM((CHUNK, C), jnp.float32),
            pltpu.VMEM((CHUNK, C), jnp.float32),
            pltpu.VMEM((CHUNK, C), jnp.float32),
            pltpu.VMEM_SHARED((NPAD, C), jnp.float32),
            pltpu.VMEM_SHARED((NPAD,), jnp.float32),
            pltpu.SemaphoreType.DMA,
            pltpu.SemaphoreType.DMA,
            pltpu.SemaphoreType.DMA,
            pltpu.SemaphoreType.DMA,
            pltpu.SemaphoreType.DMA,
            pltpu.SemaphoreType.DMA,
            pltpu.SemaphoreType.DMA,
            pltpu.SemaphoreType.DMA,
            pltpu.SemaphoreType.DMA,
        ],
    )
    def edge_kernel(src_hbm, dst_hbm, gmax_hbm, als_hbm, ald_hbm, h_hbm,
                    acc_out, den_out,
                    als_v, ald_v, gmax_v, src_all, dst_all,
                    w0, w1, w2, r0, r1, r2, acc_s, den_s,
                    sg0, sg1, sg2, ss0, ss1, ss2, sd0, sd1, sd2):
        wbufs = (w0, w1, w2)
        rbufs = (r0, r1, r2)
        sgs = (sg0, sg1, sg2)
        sss = (ss0, ss1, ss2)
        sds = (sd0, sd1, sd2)
        cid = lax.axis_index("c")
        sid = lax.axis_index("s")
        wid = sid * 2 + cid

        pltpu.sync_copy(als_hbm, als_v)
        pltpu.sync_copy(ald_hbm, ald_v)
        pltpu.sync_copy(gmax_hbm, gmax_v)
        chunk0 = wid * n_chunks
        pltpu.sync_copy(src_hbm.at[pl.ds(chunk0, n_chunks)], src_all)
        pltpu.sync_copy(dst_hbm.at[pl.ds(chunk0, n_chunks)], dst_all)

        z16 = jnp.zeros((16,), jnp.float32)

        def zrow(r, carry):
            for c in range(C // 16):
                r0[r, pl.ds(c * 16, 16)] = z16
            return carry

        lax.fori_loop(0, CHUNK, zrow, 0)

        def zw(r, carry):
            w0[pl.ds(r * 16, 16)] = z16
            return carry

        lax.fori_loop(0, CHUNK // 16, zw, 0)

        row0 = sid * rpt
        for b in range(rpt // CHUNK):
            pltpu.sync_copy(r0, acc_s.at[pl.ds(row0 + b * CHUNK, CHUNK)])
            pltpu.sync_copy(w0, den_s.at[pl.ds(row0 + b * CHUNK, CHUNK)])
        plsc.subcore_barrier()

        gv = gmax_v[...]

        def gather_start(j, b):
            pltpu.async_copy(h_hbm.at[src_all.at[j]], rbufs[b], sgs[b])

        def gather_wait(j, b):
            pltpu.make_async_copy(h_hbm.at[src_all.at[j]], rbufs[b],
                                  sgs[b]).wait()

        def scatter_start(j, b):
            pltpu.async_copy(rbufs[b], acc_s.at[dst_all.at[j]], sss[b],
                             add=True)
            pltpu.async_copy(wbufs[b], den_s.at[dst_all.at[j]], sds[b],
                             add=True)

        def scatter_wait(j, b):
            pltpu.make_async_copy(rbufs[b], acc_s.at[dst_all.at[j]],
                                  sss[b]).wait()
            pltpu.make_async_copy(wbufs[b], den_s.at[dst_all.at[j]],
                                  sds[b]).wait()

        def scale(j, b):
            rows = rbufs[b]
            wv = wbufs[b]
            for jj in range(CHUNK // 16):
                s16 = src_all[j, pl.ds(jj * 16, 16)]
                d16 = dst_all[j, pl.ds(jj * 16, 16)]
                e = plsc.load_gather(als_v, [s16]) + plsc.load_gather(ald_v, [d16])
                e = jnp.where(e > 0.0, e, e * 0.2) - gv
                w16 = jnp.exp(e)
                wv[pl.ds(jj * 16, 16)] = w16
                for l in range(16):
                    wb = _lane_bcast(w16, l)
                    r = jj * 16 + l
                    for c in range(C // 16):
                        rows[r, pl.ds(c * 16, 16)] = (
                            rows[r, pl.ds(c * 16, 16)] * wb)

        gather_start(0, 0)

        def body(i3, carry):
            for b in range(3):
                j = i3 * 3 + b
                bn = (b + 1) % 3
                gather_wait(j, b)
                if b == 2:
                    scatter_wait(j - 2, bn)
                    @pl.when(i3 < n3 - 1)
                    def _():
                        gather_start(j + 1, bn)
                else:
                    @pl.when(i3 >= 1)
                    def _():
                        scatter_wait(j - 2, bn)
                    gather_start(j + 1, bn)
                scale(j, b)
                scatter_start(j, b)
            return carry

        lax.fori_loop(0, n3, body, 0)
        for j in (n_chunks - 2, n_chunks - 1):
            scatter_wait(j, j % 3)
        plsc.subcore_barrier()

        out0 = cid * NPAD + row0
        pltpu.sync_copy(acc_s.at[pl.ds(row0, rpt)], acc_out.at[pl.ds(out0, rpt)])
        pltpu.sync_copy(den_s.at[pl.ds(row0, rpt)], den_out.at[pl.ds(out0, rpt)])

    return edge_kernel


def _gat_layer(edge_args, hext, mx, C, cols):
    srcf, dstf, ep = edge_args
    htab = hext[:, :C]
    als = hext[:, cols[0]].at[N:].set(-1e30)
    ald = hext[:, cols[1]]
    m = mx[0, cols[0]] + mx[0, cols[1]]
    g = jnp.where(m > 0.0, m, 0.2 * m)
    gvec = jnp.full((16,), g, jnp.float32)
    ek = _make_edge_kernel(C, ep)
    accf, denf = ek(srcf, dstf, gvec, als, ald, htab)
    return accf.reshape(2, NPAD, C), denf.reshape(2, NPAD, 1)


def kernel(x, edge_index, W1, a_src1, a_dst1, b1, W2, a_src2, a_dst2, b2):
    f32 = jnp.float32
    src = edge_index[0].astype(jnp.int32)
    dst = edge_index[1].astype(jnp.int32)
    loop = jnp.arange(N, dtype=jnp.int32)
    etot = src.shape[0] + N
    grain = NWORKERS * CHUNK * 3
    ep = -(-etot // grain) * grain
    padlen = ep - etot
    padv = jnp.full((padlen,), N, jnp.int32)
    srcf = jnp.concatenate([src, loop, padv]).reshape(ep // CHUNK, CHUNK)
    dstf = jnp.concatenate([dst, loop, padv]).reshape(ep // CHUNK, CHUNK)
    edge_args = (srcf, dstf, ep)

    d_in = x.shape[1]
    hid = W1.shape[1]
    out_ch = W2.shape[1]

    xp = jnp.zeros((NPAD, d_in), f32).at[:N].set(x.astype(f32))
    w1e = jnp.concatenate(
        [W1, (W1 @ a_src1[0])[:, None], (W1 @ a_dst1[0])[:, None],
         jnp.zeros((d_in, d_in - hid - 2), f32)], axis=1)
    hext1, mx1 = _proj(xp, w1e)
    acc1, den1 = _gat_layer(edge_args, hext1, mx1, hid, (hid, hid + 1))

    w2cols = max(out_ch + 2, 8)
    w2e = jnp.concatenate(
        [W2, (W2 @ a_src2[0])[:, None], (W2 @ a_dst2[0])[:, None],
         jnp.zeros((hid, w2cols - out_ch - 2), f32)], axis=1)
    hext2, mx2 = _combine_proj(acc1, den1, b1.reshape(1, hid), w2e)
    acc2, den2 = _gat_layer(edge_args, hext2, mx2, out_ch, (out_ch, out_ch + 1))

    out = _final(acc2, den2, b2.reshape(1, out_ch))
    return out[:N]


# speedup vs baseline: 64.1040x; 1.0884x over previous
import functools

import jax
import jax.numpy as jnp
from jax import lax
from jax.experimental import pallas as pl
from jax.experimental.pallas import tpu as pltpu
from jax.experimental.pallas import tpu_sc as plsc

N = 10000
NPAD = 10240
CHUNK = 128
NWORKERS = 32
ROWBLK = 1024

_GATHER_DNUMS = lax.GatherDimensionNumbers(
    offset_dims=(), collapsed_slice_dims=(0,), start_index_map=(0,))


def _lane_bcast(v16, l):
    idx = jnp.full((16, 1), l, dtype=jnp.int32)
    return lax.gather(v16, idx, dimension_numbers=_GATHER_DNUMS,
                      slice_sizes=(1,),
                      mode=lax.GatherScatterMode.PROMISE_IN_BOUNDS)


def _split_out(i, o, c, h_ref, als_ref, ald_ref, mx_ref):
    rows = i * ROWBLK + lax.broadcasted_iota(jnp.int32, (ROWBLK, 1), 0)
    valid = rows < N
    o = jnp.where(valid, o, 0.0)
    h_ref[...] = o[:, :c]
    als_ref[...] = jnp.where(valid, o[:, c:c + 1], -1e30)
    ald_ref[...] = o[:, c + 1:c + 2]

    @pl.when(i == 0)
    def _():
        mx_ref[...] = jnp.full_like(mx_ref, -1e30)

    mx_ref[...] = jnp.maximum(mx_ref[...], jnp.max(o, axis=0, keepdims=True))


def _proj_body(x_ref, w_ref, h_ref, als_ref, ald_ref, mx_ref):
    o = jnp.dot(x_ref[...], w_ref[...], preferred_element_type=jnp.float32)
    _split_out(pl.program_id(0), o, h_ref.shape[1],
               h_ref, als_ref, ald_ref, mx_ref)


def _proj(x, wext, c):
    k = x.shape[1]
    m = wext.shape[1]
    grid = NPAD // ROWBLK
    return pl.pallas_call(
        _proj_body,
        grid=(grid,),
        in_specs=[pl.BlockSpec((ROWBLK, k), lambda i: (i, 0)),
                  pl.BlockSpec((k, m), lambda i: (0, 0))],
        out_specs=[pl.BlockSpec((ROWBLK, c), lambda i: (i, 0)),
                   pl.BlockSpec((ROWBLK, 1), lambda i: (i, 0)),
                   pl.BlockSpec((ROWBLK, 1), lambda i: (i, 0)),
                   pl.BlockSpec((1, m), lambda i: (0, 0))],
        out_shape=[jax.ShapeDtypeStruct((NPAD, c), jnp.float32),
                   jax.ShapeDtypeStruct((NPAD, 1), jnp.float32),
                   jax.ShapeDtypeStruct((NPAD, 1), jnp.float32),
                   jax.ShapeDtypeStruct((1, m), jnp.float32)],
    )(x, wext)


def _combine_proj_body(acc_ref, den_ref, b_ref, w_ref,
                       h_ref, als_ref, ald_ref, mx_ref):
    a = acc_ref[0] + acc_ref[1]
    dn = den_ref[0] + den_ref[1]
    o1 = a / (dn + 1e-16) + b_ref[...]
    h2 = jnp.where(o1 > 0.0, o1, jnp.exp(o1) - 1.0)
    o = jnp.dot(h2, w_ref[...], preferred_element_type=jnp.float32)
    _split_out(pl.program_id(0), o, h_ref.shape[1],
               h_ref, als_ref, ald_ref, mx_ref)


def _combine_proj(acc, den3, b, wext, c2):
    c = acc.shape[2]
    m = wext.shape[1]
    grid = NPAD // ROWBLK
    return pl.pallas_call(
        _combine_proj_body,
        grid=(grid,),
        in_specs=[pl.BlockSpec((2, ROWBLK, c), lambda i: (0, i, 0)),
                  pl.BlockSpec((2, ROWBLK, 1), lambda i: (0, i, 0)),
                  pl.BlockSpec((1, c), lambda i: (0, 0)),
                  pl.BlockSpec((c, m), lambda i: (0, 0))],
        out_specs=[pl.BlockSpec((ROWBLK, c2), lambda i: (i, 0)),
                   pl.BlockSpec((ROWBLK, 1), lambda i: (i, 0)),
                   pl.BlockSpec((ROWBLK, 1), lambda i: (i, 0)),
                   pl.BlockSpec((1, m), lambda i: (0, 0))],
        out_shape=[jax.ShapeDtypeStruct((NPAD, c2), jnp.float32),
                   jax.ShapeDtypeStruct((NPAD, 1), jnp.float32),
                   jax.ShapeDtypeStruct((NPAD, 1), jnp.float32),
                   jax.ShapeDtypeStruct((1, m), jnp.float32)],
    )(acc, den3, b, wext)


def _final_body(acc_ref, den_ref, b_ref, o_ref):
    a = acc_ref[0] + acc_ref[1]
    dn = den_ref[0] + den_ref[1]
    o_ref[...] = a / (dn + 1e-16) + b_ref[...]


def _final(acc, den3, b):
    c = acc.shape[2]
    grid = NPAD // ROWBLK
    return pl.pallas_call(
        _final_body,
        grid=(grid,),
        in_specs=[pl.BlockSpec((2, ROWBLK, c), lambda i: (0, i, 0)),
                  pl.BlockSpec((2, ROWBLK, 1), lambda i: (0, i, 0)),
                  pl.BlockSpec((1, c), lambda i: (0, 0))],
        out_specs=pl.BlockSpec((ROWBLK, c), lambda i: (i, 0)),
        out_shape=jax.ShapeDtypeStruct((N, c), jnp.float32),
    )(acc, den3, b)


def _make_edge_kernel(C, EP, CO, spmem_h):
    n_chunks = EP // CHUNK // NWORKERS
    n3 = n_chunks // 3
    rpt = NPAD // 16
    mesh = plsc.VectorSubcoreMesh(core_axis_name="c", subcore_axis_name="s")

    @functools.partial(
        pl.kernel,
        mesh=mesh,
        compiler_params=pltpu.CompilerParams(needs_layout_passes=False,
                                             use_tc_tiling_on_sc=False),
        out_type=[jax.ShapeDtypeStruct((2 * NPAD, C), jnp.float32),
                  jax.ShapeDtypeStruct((2 * NPAD,), jnp.float32)],
        scratch_types=[
            pltpu.VMEM((NPAD,), jnp.float32),
            pltpu.VMEM((NPAD,), jnp.float32),
            pltpu.VMEM((16,), jnp.float32),
            pltpu.VMEM((n_chunks, CHUNK), jnp.int32),
            pltpu.VMEM((n_chunks, CHUNK), jnp.int32),
            pltpu.VMEM((CHUNK,), jnp.float32),
            pltpu.VMEM((CHUNK,), jnp.float32),
            pltpu.VMEM((CHUNK,), jnp.float32),
            pltpu.VMEM((CHUNK, C), jnp.float32),
            pltpu.VMEM((CHUNK, C), jnp.float32),
            pltpu.VMEM((CHUNK, C), jnp.float32),
            pltpu.VMEM_SHARED((NPAD, C), jnp.float32),
            pltpu.VMEM_SHARED((NPAD,), jnp.float32),
        ] + ([pltpu.VMEM_SHARED((NPAD, C), jnp.float32)] if spmem_h else []) + [
            pltpu.SemaphoreType.DMA,
            pltpu.SemaphoreType.DMA,
            pltpu.SemaphoreType.DMA,
            pltpu.SemaphoreType.DMA,
            pltpu.SemaphoreType.DMA,
            pltpu.SemaphoreType.DMA,
            pltpu.SemaphoreType.DMA,
            pltpu.SemaphoreType.DMA,
            pltpu.SemaphoreType.DMA,
        ],
    )
    def edge_kernel(src_hbm, dst_hbm, mx_hbm, als_hbm, ald_hbm, h_hbm,
                    acc_out, den_out,
                    als_v, ald_v, gmax_v, src_all, dst_all,
                    w0, w1, w2, r0, r1, r2, acc_s, den_s, *tail):
        if spmem_h:
            h_s = tail[0]
            tail = tail[1:]
        sg0, sg1, sg2, ss0, ss1, ss2, sd0, sd1, sd2 = tail
        wbufs = (w0, w1, w2)
        rbufs = (r0, r1, r2)
        sgs = (sg0, sg1, sg2)
        sss = (ss0, ss1, ss2)
        sds = (sd0, sd1, sd2)
        cid = lax.axis_index("c")
        sid = lax.axis_index("s")
        wid = sid * 2 + cid

        pltpu.sync_copy(als_hbm, als_v)
        pltpu.sync_copy(ald_hbm, ald_v)
        pltpu.sync_copy(mx_hbm.at[0, pl.ds(CO, 16)], gmax_v)
        chunk0 = wid * n_chunks
        pltpu.sync_copy(src_hbm.at[pl.ds(chunk0, n_chunks)], src_all)
        pltpu.sync_copy(dst_hbm.at[pl.ds(chunk0, n_chunks)], dst_all)

        z16 = jnp.zeros((16,), jnp.float32)

        def zrow(r, carry):
            for c in range(C // 16):
                r0[r, pl.ds(c * 16, 16)] = z16
            return carry

        lax.fori_loop(0, CHUNK, zrow, 0)

        def zw(r, carry):
            w0[pl.ds(r * 16, 16)] = z16
            return carry

        lax.fori_loop(0, CHUNK // 16, zw, 0)

        row0 = sid * rpt
        if spmem_h:
            pltpu.sync_copy(h_hbm.at[pl.ds(row0, rpt)], h_s.at[pl.ds(row0, rpt)])
        for b in range(rpt // CHUNK):
            pltpu.sync_copy(r0, acc_s.at[pl.ds(row0 + b * CHUNK, CHUNK)])
            pltpu.sync_copy(w0, den_s.at[pl.ds(row0 + b * CHUNK, CHUNK)])
        plsc.subcore_barrier()

        mv = gmax_v[...]
        m = _lane_bcast(mv, 0) + _lane_bcast(mv, 1)
        gv = jnp.where(m > 0.0, m, m * 0.2)

        h_src = h_s if spmem_h else h_hbm

        def gather_start(j, b):
            pltpu.async_copy(h_src.at[src_all.at[j]], rbufs[b], sgs[b])

        def gather_wait(j, b):
            pltpu.make_async_copy(h_src.at[src_all.at[j]], rbufs[b],
                                  sgs[b]).wait()

        def scatter_start(j, b):
            pltpu.async_copy(rbufs[b], acc_s.at[dst_all.at[j]], sss[b],
                             add=True)
            pltpu.async_copy(wbufs[b], den_s.at[dst_all.at[j]], sds[b],
                             add=True)

        def scatter_wait(j, b):
            pltpu.make_async_copy(rbufs[b], acc_s.at[dst_all.at[j]],
                                  sss[b]).wait()
            pltpu.make_async_copy(wbufs[b], den_s.at[dst_all.at[j]],
                                  sds[b]).wait()

        def scale(j, b):
            rows = rbufs[b]
            wv = wbufs[b]
            for jj in range(CHUNK // 16):
                s16 = src_all[j, pl.ds(jj * 16, 16)]
                d16 = dst_all[j, pl.ds(jj * 16, 16)]
                e = (plsc.load_gather(als_v, [s16])
                     + plsc.load_gather(ald_v, [d16]))
                e = jnp.where(e > 0.0, e, e * 0.2) - gv
                w16 = jnp.exp(e)
                wv[pl.ds(jj * 16, 16)] = w16
                for l in range(16):
                    wb = _lane_bcast(w16, l)
                    r = jj * 16 + l
                    for c in range(C // 16):
                        rows[r, pl.ds(c * 16, 16)] = (
                            rows[r, pl.ds(c * 16, 16)] * wb)

        gather_start(0, 0)

        def body(i3, carry):
            for b in range(3):
                j = i3 * 3 + b
                bn = (b + 1) % 3
                gather_wait(j, b)
                if b == 2:
                    scatter_wait(j - 2, bn)
                    @pl.when(i3 < n3 - 1)
                    def _():
                        gather_start(j + 1, bn)
                else:
                    @pl.when(i3 >= 1)
                    def _():
                        scatter_wait(j - 2, bn)
                    gather_start(j + 1, bn)
                scale(j, b)
                scatter_start(j, b)
            return carry

        lax.fori_loop(0, n3, body, 0)
        for j in (n_chunks - 2, n_chunks - 1):
            scatter_wait(j, j % 3)
        plsc.subcore_barrier()

        out0 = cid * NPAD + row0
        pltpu.sync_copy(acc_s.at[pl.ds(row0, rpt)], acc_out.at[pl.ds(out0, rpt)])
        pltpu.sync_copy(den_s.at[pl.ds(row0, rpt)], den_out.at[pl.ds(out0, rpt)])

    return edge_kernel


def _gat_layer(edge_args, htab, als, ald, mx, co, spmem_h):
    srcf, dstf, ep = edge_args
    c = htab.shape[1]
    ek = _make_edge_kernel(c, ep, co, spmem_h)
    accf, denf = ek(srcf, dstf, mx, als.reshape(NPAD), ald.reshape(NPAD),
                    htab)
    return accf.reshape(2, NPAD, c), denf.reshape(2, NPAD, 1)


def kernel(x, edge_index, W1, a_src1, a_dst1, b1, W2, a_src2, a_dst2, b2):
    f32 = jnp.float32
    src = edge_index[0].astype(jnp.int32)
    dst = edge_index[1].astype(jnp.int32)
    loop = jnp.arange(N, dtype=jnp.int32)
    etot = src.shape[0] + N
    grain = NWORKERS * CHUNK * 3
    ep = -(-etot // grain) * grain
    padlen = ep - etot
    padv = jnp.full((padlen,), N, jnp.int32)
    srcf = jnp.concatenate([src, loop, padv]).reshape(ep // CHUNK, CHUNK)
    dstf = jnp.concatenate([dst, loop, padv]).reshape(ep // CHUNK, CHUNK)
    edge_args = (srcf, dstf, ep)

    d_in = x.shape[1]
    hid = W1.shape[1]
    out_ch = W2.shape[1]

    w1e = jnp.concatenate(
        [W1, (W1 @ a_src1[0])[:, None], (W1 @ a_dst1[0])[:, None],
         jnp.zeros((d_in, d_in - hid - 2), f32)], axis=1)
    htab1, als1, ald1, mx1 = _proj(x.astype(f32), w1e, hid)
    acc1, den1 = _gat_layer(edge_args, htab1, als1, ald1, mx1, hid, False)

    m2 = max(2 * out_ch, 32)
    w2e = jnp.concatenate(
        [W2, (W2 @ a_src2[0])[:, None], (W2 @ a_dst2[0])[:, None],
         jnp.zeros((hid, m2 - out_ch - 2), f32)], axis=1)
    htab2, als2, ald2, mx2 = _combine_proj(acc1, den1, b1.reshape(1, hid),
                                           w2e, out_ch)
    acc2, den2 = _gat_layer(edge_args, htab2, als2, ald2, mx2, out_ch, True)

    return _final(acc2, den2, b2.reshape(1, out_ch))


# speedup vs baseline: 68.4835x; 1.0683x over previous
import functools

import jax
import jax.numpy as jnp
from jax import lax
from jax.experimental import pallas as pl
from jax.experimental.pallas import tpu as pltpu
from jax.experimental.pallas import tpu_sc as plsc

N = 10000
NPAD = 10240
CHUNK = 128
NWORKERS = 32
ROWBLK = 1024
AROWS = NPAD // 128

_GATHER_DNUMS = lax.GatherDimensionNumbers(
    offset_dims=(), collapsed_slice_dims=(0,), start_index_map=(0,))


def _lane_bcast(v16, l):
    idx = jnp.full((16, 1), l, dtype=jnp.int32)
    return lax.gather(v16, idx, dimension_numbers=_GATHER_DNUMS,
                      slice_sizes=(1,),
                      mode=lax.GatherScatterMode.PROMISE_IN_BOUNDS)


def _split_out(i, o, c, h_ref, als_ref, ald_ref, mx_ref):
    rows = i * ROWBLK + lax.broadcasted_iota(jnp.int32, (ROWBLK, 1), 0)
    valid = rows < N
    o = jnp.where(valid, o, 0.0)
    h_ref[...] = o[:, :c]
    als = jnp.where(valid, o[:, c:c + 1], -1e30)
    als_ref[...] = als.reshape(ROWBLK // 128, 128)
    ald_ref[...] = o[:, c + 1:c + 2].reshape(ROWBLK // 128, 128)

    @pl.when(i == 0)
    def _():
        mx_ref[...] = jnp.full_like(mx_ref, -1e30)

    mx_ref[...] = jnp.maximum(mx_ref[...], jnp.max(o, axis=0, keepdims=True))


def _aa_specs():
    return ([pl.BlockSpec((ROWBLK // 128, 128), lambda i: (i, 0)),
             pl.BlockSpec((ROWBLK // 128, 128), lambda i: (i, 0))],
            [jax.ShapeDtypeStruct((AROWS, 128), jnp.float32),
             jax.ShapeDtypeStruct((AROWS, 128), jnp.float32)])


def _proj_body(x_ref, w_ref, h_ref, als_ref, ald_ref, mx_ref):
    o = jnp.dot(x_ref[...], w_ref[...], preferred_element_type=jnp.float32)
    _split_out(pl.program_id(0), o, h_ref.shape[1],
               h_ref, als_ref, ald_ref, mx_ref)


def _proj(x, wext, c):
    k = x.shape[1]
    m = wext.shape[1]
    grid = NPAD // ROWBLK
    aa_specs, aa_shapes = _aa_specs()
    return pl.pallas_call(
        _proj_body,
        grid=(grid,),
        in_specs=[pl.BlockSpec((ROWBLK, k), lambda i: (i, 0)),
                  pl.BlockSpec((k, m), lambda i: (0, 0))],
        out_specs=[pl.BlockSpec((ROWBLK, c), lambda i: (i, 0))] + aa_specs
                  + [pl.BlockSpec((1, m), lambda i: (0, 0))],
        out_shape=[jax.ShapeDtypeStruct((NPAD, c), jnp.float32)] + aa_shapes
                  + [jax.ShapeDtypeStruct((1, m), jnp.float32)],
    )(x, wext)


def _combine_proj_body(acc_ref, den_ref, b_ref, w_ref,
                       h_ref, als_ref, ald_ref, mx_ref):
    a = acc_ref[0] + acc_ref[1]
    dn = den_ref[0] + den_ref[1]
    o1 = a / (dn + 1e-16) + b_ref[...]
    h2 = jnp.where(o1 > 0.0, o1, jnp.exp(o1) - 1.0)
    o = jnp.dot(h2, w_ref[...], preferred_element_type=jnp.float32)
    _split_out(pl.program_id(0), o, h_ref.shape[1],
               h_ref, als_ref, ald_ref, mx_ref)


def _combine_proj(acc, den3, b, wext, c2):
    c = acc.shape[2]
    m = wext.shape[1]
    grid = NPAD // ROWBLK
    aa_specs, aa_shapes = _aa_specs()
    return pl.pallas_call(
        _combine_proj_body,
        grid=(grid,),
        in_specs=[pl.BlockSpec((2, ROWBLK, c), lambda i: (0, i, 0)),
                  pl.BlockSpec((2, ROWBLK, 1), lambda i: (0, i, 0)),
                  pl.BlockSpec((1, c), lambda i: (0, 0)),
                  pl.BlockSpec((c, m), lambda i: (0, 0))],
        out_specs=[pl.BlockSpec((ROWBLK, c2), lambda i: (i, 0))] + aa_specs
                  + [pl.BlockSpec((1, m), lambda i: (0, 0))],
        out_shape=[jax.ShapeDtypeStruct((NPAD, c2), jnp.float32)] + aa_shapes
                  + [jax.ShapeDtypeStruct((1, m), jnp.float32)],
    )(acc, den3, b, wext)


def _final_body(acc_ref, den_ref, b_ref, o_ref):
    a = acc_ref[0] + acc_ref[1]
    dn = den_ref[0] + den_ref[1]
    o_ref[...] = a / (dn + 1e-16) + b_ref[...]


def _final(acc, den3, b):
    c = acc.shape[2]
    grid = NPAD // ROWBLK
    return pl.pallas_call(
        _final_body,
        grid=(grid,),
        in_specs=[pl.BlockSpec((2, ROWBLK, c), lambda i: (0, i, 0)),
                  pl.BlockSpec((2, ROWBLK, 1), lambda i: (0, i, 0)),
                  pl.BlockSpec((1, c), lambda i: (0, 0))],
        out_specs=pl.BlockSpec((ROWBLK, c), lambda i: (i, 0)),
        out_shape=jax.ShapeDtypeStruct((N, c), jnp.float32),
    )(acc, den3, b)


def _make_edge_kernel(C, EP, CO, spmem_h):
    n_chunks = EP // CHUNK // NWORKERS
    n3 = n_chunks // 3
    rpt = NPAD // 16
    mesh = plsc.VectorSubcoreMesh(core_axis_name="c", subcore_axis_name="s")

    @functools.partial(
        pl.kernel,
        mesh=mesh,
        compiler_params=pltpu.CompilerParams(needs_layout_passes=False,
                                             use_tc_tiling_on_sc=False),
        out_type=[jax.ShapeDtypeStruct((2 * NPAD, C), jnp.float32),
                  jax.ShapeDtypeStruct((2 * NPAD,), jnp.float32)],
        scratch_types=[
            pltpu.VMEM((AROWS, 128), jnp.float32),
            pltpu.VMEM((AROWS, 128), jnp.float32),
            pltpu.VMEM((16,), jnp.float32),
            pltpu.VMEM((n_chunks, CHUNK), jnp.int32),
            pltpu.VMEM((n_chunks, CHUNK), jnp.int32),
            pltpu.VMEM((CHUNK,), jnp.float32),
            pltpu.VMEM((CHUNK,), jnp.float32),
            pltpu.VMEM((CHUNK,), jnp.float32),
            pltpu.VMEM((CHUNK, C), jnp.float32),
            pltpu.VMEM((CHUNK, C), jnp.float32),
            pltpu.VMEM((CHUNK, C), jnp.float32),
            pltpu.VMEM_SHARED((NPAD, C), jnp.float32),
            pltpu.VMEM_SHARED((NPAD,), jnp.float32),
        ] + ([pltpu.VMEM_SHARED((NPAD, C), jnp.float32)] if spmem_h else [])
        + [
            pltpu.SemaphoreType.DMA,
            pltpu.SemaphoreType.DMA,
            pltpu.SemaphoreType.DMA,
            pltpu.SemaphoreType.DMA,
            pltpu.SemaphoreType.DMA,
            pltpu.SemaphoreType.DMA,
            pltpu.SemaphoreType.DMA,
            pltpu.SemaphoreType.DMA,
            pltpu.SemaphoreType.DMA,
        ],
    )
    def edge_kernel(pk_hbm, mx_hbm, als_hbm, ald_hbm, h_hbm,
                    acc_out, den_out,
                    als_v, ald_v, gmax_v, src_all, dst_all,
                    w0, w1, w2, r0, r1, r2, acc_s, den_s, *tail):
        if spmem_h:
            h_s = tail[0]
            tail = tail[1:]
        sg0, sg1, sg2, ss0, ss1, ss2, sd0, sd1, sd2 = tail
        wbufs = (w0, w1, w2)
        rbufs = (r0, r1, r2)
        sgs = (sg0, sg1, sg2)
        sss = (ss0, ss1, ss2)
        sds = (sd0, sd1, sd2)
        cid = lax.axis_index("c")
        sid = lax.axis_index("s")
        wid = sid * 2 + cid

        pltpu.sync_copy(als_hbm, als_v)
        pltpu.sync_copy(ald_hbm, ald_v)
        pltpu.sync_copy(mx_hbm.at[0, pl.ds(CO, 16)], gmax_v)
        chunk0 = wid * n_chunks
        pltpu.sync_copy(pk_hbm.at[pl.ds(chunk0, n_chunks)], src_all)

        def unpack(g, carry):
            ch = g // 8
            off = (g % 8) * 16
            v = src_all[ch, pl.ds(off, 16)]
            dst_all[ch, pl.ds(off, 16)] = lax.shift_right_logical(v, 16)
            src_all[ch, pl.ds(off, 16)] = lax.bitwise_and(v, 0xFFFF)
            return carry

        lax.fori_loop(0, n_chunks * 8, unpack, 0)

        z16 = jnp.zeros((16,), jnp.float32)
        row0 = sid * rpt
        if spmem_h:
            pltpu.sync_copy(h_hbm.at[pl.ds(row0, rpt)], h_s.at[pl.ds(row0, rpt)])

        def zrow(r, carry):
            for c in range(C // 16):
                r0[r, pl.ds(c * 16, 16)] = z16
            return carry

        lax.fori_loop(0, CHUNK, zrow, 0)

        def zw(r, carry):
            w0[pl.ds(r * 16, 16)] = z16
            return carry

        lax.fori_loop(0, CHUNK // 16, zw, 0)

        for b in range(rpt // CHUNK):
            pltpu.sync_copy(r0, acc_s.at[pl.ds(row0 + b * CHUNK, CHUNK)])
            pltpu.sync_copy(w0, den_s.at[pl.ds(row0 + b * CHUNK, CHUNK)])
        plsc.subcore_barrier()

        mv = gmax_v[...]
        m = _lane_bcast(mv, 0) + _lane_bcast(mv, 1)
        gv = jnp.where(m > 0.0, m, m * 0.2)

        h_src = h_s if spmem_h else h_hbm

        def gather_start(j, b):
            pltpu.async_copy(h_src.at[src_all.at[j]], rbufs[b], sgs[b])

        def gather_wait(j, b):
            pltpu.make_async_copy(h_src.at[src_all.at[j]], rbufs[b],
                                  sgs[b]).wait()

        def scatter_start(j, b):
            pltpu.async_copy(rbufs[b], acc_s.at[dst_all.at[j]], sss[b],
                             add=True)
            pltpu.async_copy(wbufs[b], den_s.at[dst_all.at[j]], sds[b],
                             add=True)

        def scatter_wait(j, b):
            pltpu.make_async_copy(rbufs[b], acc_s.at[dst_all.at[j]],
                                  sss[b]).wait()
            pltpu.make_async_copy(wbufs[b], den_s.at[dst_all.at[j]],
                                  sds[b]).wait()

        def scale(j, b):
            rows = rbufs[b]
            wv = wbufs[b]
            for jj in range(CHUNK // 16):
                s16 = src_all[j, pl.ds(jj * 16, 16)]
                d16 = dst_all[j, pl.ds(jj * 16, 16)]
                e = (plsc.load_gather(
                        als_v, [lax.shift_right_logical(s16, 7),
                                lax.bitwise_and(s16, 127)])
                     + plsc.load_gather(
                        ald_v, [lax.shift_right_logical(d16, 7),
                                lax.bitwise_and(d16, 127)]))
                e = jnp.where(e > 0.0, e, e * 0.2) - gv
                w16 = jnp.exp(e)
                wv[pl.ds(jj * 16, 16)] = w16
                for l in range(16):
                    wb = _lane_bcast(w16, l)
                    r = jj * 16 + l
                    for c in range(C // 16):
                        rows[r, pl.ds(c * 16, 16)] = (
                            rows[r, pl.ds(c * 16, 16)] * wb)

        gather_start(0, 0)

        def body(i3, carry):
            for b in range(3):
                j = i3 * 3 + b
                bn = (b + 1) % 3
                gather_wait(j, b)
                if b == 2:
                    scatter_wait(j - 2, bn)
                    @pl.when(i3 < n3 - 1)
                    def _():
                        gather_start(j + 1, bn)
                else:
                    @pl.when(i3 >= 1)
                    def _():
                        scatter_wait(j - 2, bn)
                    gather_start(j + 1, bn)
                scale(j, b)
                scatter_start(j, b)
            return carry

        lax.fori_loop(0, n3, body, 0)
        for j in (n_chunks - 2, n_chunks - 1):
            scatter_wait(j, j % 3)
        plsc.subcore_barrier()

        out0 = cid * NPAD + row0
        pltpu.sync_copy(acc_s.at[pl.ds(row0, rpt)], acc_out.at[pl.ds(out0, rpt)])
        pltpu.sync_copy(den_s.at[pl.ds(row0, rpt)], den_out.at[pl.ds(out0, rpt)])

    return edge_kernel


def _gat_layer(edge_args, htab, als, ald, mx, co, spmem_h):
    pkf, ep = edge_args
    c = htab.shape[1]
    ek = _make_edge_kernel(c, ep, co, spmem_h)
    accf, denf = ek(pkf, mx, als, ald, htab)
    return accf.reshape(2, NPAD, c), denf.reshape(2, NPAD, 1)


def kernel(x, edge_index, W1, a_src1, a_dst1, b1, W2, a_src2, a_dst2, b2):
    f32 = jnp.float32
    src = edge_index[0].astype(jnp.int32)
    dst = edge_index[1].astype(jnp.int32)
    loop = jnp.arange(N, dtype=jnp.int32)
    etot = src.shape[0] + N
    grain = NWORKERS * CHUNK * 3
    ep = -(-etot // grain) * grain
    padlen = ep - etot
    pk_pad = jnp.full((padlen,), N + (N << 16), jnp.int32)
    pkf = jnp.concatenate(
        [src + (dst << 16), loop + (loop << 16), pk_pad],
    ).reshape(ep // CHUNK, CHUNK)
    edge_args = (pkf, ep)

    d_in = x.shape[1]
    hid = W1.shape[1]
    out_ch = W2.shape[1]

    w1e = jnp.concatenate(
        [W1, (W1 @ a_src1[0])[:, None], (W1 @ a_dst1[0])[:, None],
         jnp.zeros((d_in, d_in - hid - 2), f32)], axis=1)
    htab1, als1, ald1, mx1 = _proj(x.astype(f32), w1e, hid)
    acc1, den1 = _gat_layer(edge_args, htab1, als1, ald1, mx1, hid, False)

    m2 = max(2 * out_ch, 32)
    w2e = jnp.concatenate(
        [W2, (W2 @ a_src2[0])[:, None], (W2 @ a_dst2[0])[:, None],
         jnp.zeros((hid, m2 - out_ch - 2), f32)], axis=1)
    htab2, als2, ald2, mx2 = _combine_proj(acc1, den1, b1.reshape(1, hid),
                                           w2e, out_ch)
    acc2, den2 = _gat_layer(edge_args, htab2, als2, ald2, mx2, out_ch, True)

    return _final(acc2, den2, b2.reshape(1, out_ch))


# speedup vs baseline: 71.5394x; 1.0446x over previous
import functools

import jax
import jax.numpy as jnp
from jax import lax
from jax.experimental import pallas as pl
from jax.experimental.pallas import tpu as pltpu
from jax.experimental.pallas import tpu_sc as plsc

N = 10000
NPAD = 10240
CHUNK = 128
NWORKERS = 32
ROWBLK = 1024
AROWS = NPAD // 128

_GATHER_DNUMS = lax.GatherDimensionNumbers(
    offset_dims=(), collapsed_slice_dims=(0,), start_index_map=(0,))


def _lane_bcast(v16, l):
    idx = jnp.full((16, 1), l, dtype=jnp.int32)
    return lax.gather(v16, idx, dimension_numbers=_GATHER_DNUMS,
                      slice_sizes=(1,),
                      mode=lax.GatherScatterMode.PROMISE_IN_BOUNDS)


def _split_out(i, o, c, h_ref, als_ref, ald_ref, mx_ref):
    rows = i * ROWBLK + lax.broadcasted_iota(jnp.int32, (ROWBLK, 1), 0)
    valid = rows < N
    o = jnp.where(valid, o, 0.0)
    h_ref[...] = o[:, :c].astype(h_ref.dtype)
    als = jnp.where(valid, o[:, c:c + 1], -1e30)
    als_ref[...] = als.reshape(ROWBLK // 128, 128)
    ald_ref[...] = o[:, c + 1:c + 2].reshape(ROWBLK // 128, 128)

    @pl.when(i == 0)
    def _():
        mx_ref[...] = jnp.full_like(mx_ref, -1e30)

    mx_ref[...] = jnp.maximum(mx_ref[...], jnp.max(o, axis=0, keepdims=True))


def _aa_specs():
    return ([pl.BlockSpec((ROWBLK // 128, 128), lambda i: (i, 0)),
             pl.BlockSpec((ROWBLK // 128, 128), lambda i: (i, 0))],
            [jax.ShapeDtypeStruct((AROWS, 128), jnp.float32),
             jax.ShapeDtypeStruct((AROWS, 128), jnp.float32)])


def _proj_body(x_ref, w_ref, h_ref, als_ref, ald_ref, mx_ref):
    o = jnp.dot(x_ref[...], w_ref[...], preferred_element_type=jnp.float32)
    _split_out(pl.program_id(0), o, h_ref.shape[1],
               h_ref, als_ref, ald_ref, mx_ref)


def _proj(x, wext, c, h_dtype):
    k = x.shape[1]
    m = wext.shape[1]
    grid = NPAD // ROWBLK
    aa_specs, aa_shapes = _aa_specs()
    return pl.pallas_call(
        _proj_body,
        grid=(grid,),
        in_specs=[pl.BlockSpec((ROWBLK, k), lambda i: (i, 0)),
                  pl.BlockSpec((k, m), lambda i: (0, 0))],
        out_specs=[pl.BlockSpec((ROWBLK, c), lambda i: (i, 0))] + aa_specs
                  + [pl.BlockSpec((1, m), lambda i: (0, 0))],
        out_shape=[jax.ShapeDtypeStruct((NPAD, c), h_dtype)] + aa_shapes
                  + [jax.ShapeDtypeStruct((1, m), jnp.float32)],
    )(x, wext)


def _combine_proj_body(acc_ref, den_ref, b_ref, w_ref,
                       h_ref, als_ref, ald_ref, mx_ref):
    a = acc_ref[0] + acc_ref[1]
    dn = den_ref[0] + den_ref[1]
    o1 = a / (dn + 1e-16) + b_ref[...]
    h2 = jnp.where(o1 > 0.0, o1, jnp.exp(o1) - 1.0)
    o = jnp.dot(h2, w_ref[...], preferred_element_type=jnp.float32)
    _split_out(pl.program_id(0), o, h_ref.shape[1],
               h_ref, als_ref, ald_ref, mx_ref)


def _combine_proj(acc, den3, b, wext, c2):
    c = acc.shape[2]
    m = wext.shape[1]
    grid = NPAD // ROWBLK
    aa_specs, aa_shapes = _aa_specs()
    return pl.pallas_call(
        _combine_proj_body,
        grid=(grid,),
        in_specs=[pl.BlockSpec((2, ROWBLK, c), lambda i: (0, i, 0)),
                  pl.BlockSpec((2, ROWBLK, 1), lambda i: (0, i, 0)),
                  pl.BlockSpec((1, c), lambda i: (0, 0)),
                  pl.BlockSpec((c, m), lambda i: (0, 0))],
        out_specs=[pl.BlockSpec((ROWBLK, c2), lambda i: (i, 0))] + aa_specs
                  + [pl.BlockSpec((1, m), lambda i: (0, 0))],
        out_shape=[jax.ShapeDtypeStruct((NPAD, c2), jnp.float32)] + aa_shapes
                  + [jax.ShapeDtypeStruct((1, m), jnp.float32)],
    )(acc, den3, b, wext)


def _final_body(acc_ref, den_ref, b_ref, o_ref):
    a = acc_ref[0] + acc_ref[1]
    dn = den_ref[0] + den_ref[1]
    o_ref[...] = a / (dn + 1e-16) + b_ref[...]


def _final(acc, den3, b):
    c = acc.shape[2]
    grid = NPAD // ROWBLK
    return pl.pallas_call(
        _final_body,
        grid=(grid,),
        in_specs=[pl.BlockSpec((2, ROWBLK, c), lambda i: (0, i, 0)),
                  pl.BlockSpec((2, ROWBLK, 1), lambda i: (0, i, 0)),
                  pl.BlockSpec((1, c), lambda i: (0, 0))],
        out_specs=pl.BlockSpec((ROWBLK, c), lambda i: (i, 0)),
        out_shape=jax.ShapeDtypeStruct((N, c), jnp.float32),
    )(acc, den3, b)


def _make_edge_kernel(C, EP, CO, hmode):
    hdt = jnp.bfloat16 if hmode == "bf16" else jnp.float32
    n_chunks = EP // CHUNK // NWORKERS
    n3 = n_chunks // 3
    rpt = NPAD // 16
    mesh = plsc.VectorSubcoreMesh(core_axis_name="c", subcore_axis_name="s")

    @functools.partial(
        pl.kernel,
        mesh=mesh,
        compiler_params=pltpu.CompilerParams(needs_layout_passes=False,
                                             use_tc_tiling_on_sc=False),
        out_type=[jax.ShapeDtypeStruct((2 * NPAD, C), jnp.float32),
                  jax.ShapeDtypeStruct((2 * NPAD,), jnp.float32)],
        scratch_types=[
            pltpu.VMEM((AROWS, 128), jnp.float32),
            pltpu.VMEM((AROWS, 128), jnp.float32),
            pltpu.VMEM((16,), jnp.float32),
            pltpu.VMEM((n_chunks, CHUNK), jnp.int32),
            pltpu.VMEM((n_chunks, CHUNK), jnp.int32),
            pltpu.VMEM((CHUNK,), jnp.float32),
            pltpu.VMEM((CHUNK,), jnp.float32),
            pltpu.VMEM((CHUNK,), jnp.float32),
            pltpu.VMEM((CHUNK, C), hdt),
            pltpu.VMEM((CHUNK, C), hdt),
            pltpu.VMEM((CHUNK, C), hdt),
        ] + ([pltpu.VMEM((CHUNK, C), jnp.float32)] * 3 if hmode == "bf16"
             else [])
        + [
            pltpu.VMEM_SHARED((NPAD, C), jnp.float32),
            pltpu.VMEM_SHARED((NPAD,), jnp.float32),
        ] + ([pltpu.VMEM_SHARED((NPAD, C), hdt)] if hmode != "hbm" else [])
        + [
            pltpu.SemaphoreType.DMA,
            pltpu.SemaphoreType.DMA,
            pltpu.SemaphoreType.DMA,
            pltpu.SemaphoreType.DMA,
            pltpu.SemaphoreType.DMA,
            pltpu.SemaphoreType.DMA,
            pltpu.SemaphoreType.DMA,
            pltpu.SemaphoreType.DMA,
            pltpu.SemaphoreType.DMA,
        ],
    )
    def edge_kernel(pk_hbm, mx_hbm, als_hbm, ald_hbm, h_hbm,
                    acc_out, den_out,
                    als_v, ald_v, gmax_v, src_all, dst_all,
                    w0, w1, w2, r0, r1, r2, *tail):
        tail = list(tail)
        if hmode == "bf16":
            sbufs = tuple(tail[:3])
            tail = tail[3:]
        acc_s, den_s = tail[:2]
        tail = tail[2:]
        if hmode != "hbm":
            h_s = tail.pop(0)
        sg0, sg1, sg2, ss0, ss1, ss2, sd0, sd1, sd2 = tail
        wbufs = (w0, w1, w2)
        rbufs = (r0, r1, r2)
        sgs = (sg0, sg1, sg2)
        sss = (ss0, ss1, ss2)
        sds = (sd0, sd1, sd2)
        cid = lax.axis_index("c")
        sid = lax.axis_index("s")
        wid = sid * 2 + cid

        pltpu.sync_copy(als_hbm, als_v)
        pltpu.sync_copy(ald_hbm, ald_v)
        pltpu.sync_copy(mx_hbm.at[0, pl.ds(CO, 16)], gmax_v)
        chunk0 = wid * n_chunks
        pltpu.sync_copy(pk_hbm.at[pl.ds(chunk0, n_chunks)], src_all)

        def unpack(g, carry):
            ch = g // 8
            off = (g % 8) * 16
            v = src_all[ch, pl.ds(off, 16)]
            dst_all[ch, pl.ds(off, 16)] = lax.shift_right_logical(v, 16)
            src_all[ch, pl.ds(off, 16)] = lax.bitwise_and(v, 0xFFFF)
            return carry

        lax.fori_loop(0, n_chunks * 8, unpack, 0)

        z16 = jnp.zeros((16,), jnp.float32)
        row0 = sid * rpt
        if hmode != "hbm":
            pltpu.sync_copy(h_hbm.at[pl.ds(row0, rpt)], h_s.at[pl.ds(row0, rpt)])

        zb = sbufs[0] if hmode == "bf16" else r0

        def zrow(r, carry):
            for c in range(C // 16):
                zb[r, pl.ds(c * 16, 16)] = z16
            return carry

        lax.fori_loop(0, CHUNK, zrow, 0)

        def zw(r, carry):
            w0[pl.ds(r * 16, 16)] = z16
            return carry

        lax.fori_loop(0, CHUNK // 16, zw, 0)

        for b in range(rpt // CHUNK):
            pltpu.sync_copy(zb, acc_s.at[pl.ds(row0 + b * CHUNK, CHUNK)])
            pltpu.sync_copy(w0, den_s.at[pl.ds(row0 + b * CHUNK, CHUNK)])
        plsc.subcore_barrier()

        mv = gmax_v[...]
        m = _lane_bcast(mv, 0) + _lane_bcast(mv, 1)
        gv = jnp.where(m > 0.0, m, m * 0.2)

        h_src = h_hbm if hmode == "hbm" else h_s
        abufs = sbufs if hmode == "bf16" else rbufs

        def gather_start(j, b):
            pltpu.async_copy(h_src.at[src_all.at[j]], rbufs[b], sgs[b])

        def gather_wait(j, b):
            pltpu.make_async_copy(h_src.at[src_all.at[j]], rbufs[b],
                                  sgs[b]).wait()

        def scatter_start(j, b):
            pltpu.async_copy(abufs[b], acc_s.at[dst_all.at[j]], sss[b],
                             add=True)
            pltpu.async_copy(wbufs[b], den_s.at[dst_all.at[j]], sds[b],
                             add=True)

        def scatter_wait(j, b):
            pltpu.make_async_copy(abufs[b], acc_s.at[dst_all.at[j]],
                                  sss[b]).wait()
            pltpu.make_async_copy(wbufs[b], den_s.at[dst_all.at[j]],
                                  sds[b]).wait()

        def scale(j, b):
            rows = rbufs[b]
            wv = wbufs[b]
            for jj in range(CHUNK // 16):
                s16 = src_all[j, pl.ds(jj * 16, 16)]
                d16 = dst_all[j, pl.ds(jj * 16, 16)]
                e = (plsc.load_gather(
                        als_v, [lax.shift_right_logical(s16, 7),
                                lax.bitwise_and(s16, 127)])
                     + plsc.load_gather(
                        ald_v, [lax.shift_right_logical(d16, 7),
                                lax.bitwise_and(d16, 127)]))
                e = jnp.where(e > 0.0, e, e * 0.2) - gv
                w16 = jnp.exp(e)
                wv[pl.ds(jj * 16, 16)] = w16
                for l in range(16):
                    wb = _lane_bcast(w16, l)
                    r = jj * 16 + l
                    if hmode == "bf16":
                        sb = sbufs[b]
                        for g in range(C // 32):
                            hb = rows[r, pl.ds(g * 32, 32)]
                            ev, od = plsc.unpack(
                                hb, format=plsc.PackFormat.INTERLEAVED)
                            sb[r, pl.ds(g * 32, 16)] = ev * wb
                            sb[r, pl.ds(g * 32 + 16, 16)] = od * wb
                    else:
                        for c in range(C // 16):
                            rows[r, pl.ds(c * 16, 16)] = (
                                rows[r, pl.ds(c * 16, 16)] * wb)

        gather_start(0, 0)

        def body(i3, carry):
            for b in range(3):
                j = i3 * 3 + b
                bn = (b + 1) % 3
                gather_wait(j, b)
                if b == 2:
                    scatter_wait(j - 2, bn)
                    @pl.when(i3 < n3 - 1)
                    def _():
                        gather_start(j + 1, bn)
                else:
                    @pl.when(i3 >= 1)
                    def _():
                        scatter_wait(j - 2, bn)
                    gather_start(j + 1, bn)
                scale(j, b)
                scatter_start(j, b)
            return carry

        lax.fori_loop(0, n3, body, 0)
        for j in (n_chunks - 2, n_chunks - 1):
            scatter_wait(j, j % 3)
        plsc.subcore_barrier()

        out0 = cid * NPAD + row0
        pltpu.sync_copy(acc_s.at[pl.ds(row0, rpt)], acc_out.at[pl.ds(out0, rpt)])
        pltpu.sync_copy(den_s.at[pl.ds(row0, rpt)], den_out.at[pl.ds(out0, rpt)])

    return edge_kernel


def _gat_layer(edge_args, htab, als, ald, mx, co, hmode):
    pkf, ep = edge_args
    c = htab.shape[1]
    ek = _make_edge_kernel(c, ep, co, hmode)
    accf, denf = ek(pkf, mx, als, ald, htab)
    return accf.reshape(2, NPAD, c), denf.reshape(2, NPAD, 1)


def kernel(x, edge_index, W1, a_src1, a_dst1, b1, W2, a_src2, a_dst2, b2):
    f32 = jnp.float32
    src = edge_index[0].astype(jnp.int32)
    dst = edge_index[1].astype(jnp.int32)
    loop = jnp.arange(N, dtype=jnp.int32)
    etot = src.shape[0] + N
    grain = NWORKERS * CHUNK * 3
    ep = -(-etot // grain) * grain
    padlen = ep - etot
    pk_pad = jnp.full((padlen,), N + (N << 16), jnp.int32)
    pkf = jnp.concatenate(
        [src + (dst << 16), loop + (loop << 16), pk_pad],
    ).reshape(ep // CHUNK, CHUNK)
    edge_args = (pkf, ep)

    d_in = x.shape[1]
    hid = W1.shape[1]
    out_ch = W2.shape[1]

    cperm = jnp.array([32 * (p // 32) + 16 * (p % 2) + (p % 32) // 2
                       for p in range(hid)])
    w1e = jnp.concatenate(
        [W1[:, cperm], (W1 @ a_src1[0])[:, None], (W1 @ a_dst1[0])[:, None],
         jnp.zeros((d_in, d_in - hid - 2), f32)], axis=1)
    htab1, als1, ald1, mx1 = _proj(x.astype(f32), w1e, hid, jnp.float32)
    acc1, den1 = _gat_layer(edge_args, htab1, als1, ald1, mx1, hid, "hbm")

    m2 = max(2 * out_ch, 32)
    w2e = jnp.concatenate(
        [W2[cperm, :], (W2 @ a_src2[0])[cperm][:, None],
         (W2 @ a_dst2[0])[cperm][:, None],
         jnp.zeros((hid, m2 - out_ch - 2), f32)], axis=1)
    htab2, als2, ald2, mx2 = _combine_proj(acc1, den1,
                                           b1[cperm].reshape(1, hid),
                                           w2e, out_ch)
    acc2, den2 = _gat_layer(edge_args, htab2, als2, ald2, mx2, out_ch, "f32")

    return _final(acc2, den2, b2.reshape(1, out_ch))
